# baseline (speedup 1.0000x reference)
import functools

import jax
import jax.numpy as jnp
from jax import lax
from jax.experimental import pallas as pl
from jax.experimental.pallas import tpu as pltpu
from jax.experimental.pallas import tpu_sc as plsc

N = 10000
E = 320000
D = 128

NC = 2
NS = 16
NW = NC * NS
C = 128
NG = 10
NCH = 8 * NG
EPW = NCH * C
E_PAD = NW * EPW
N_PAD = 10240
RPT = N_PAD // NS

_mesh = plsc.VectorSubcoreMesh(core_axis_name="c", subcore_axis_name="s")


@functools.partial(
    pl.kernel,
    mesh=_mesh,
    out_type=jax.ShapeDtypeStruct((NC * N_PAD, D), jnp.float32),
    scratch_types=[
        pltpu.VMEM((NCH, C), jnp.int32),
        pltpu.VMEM((C, D), jnp.float32),
        pltpu.VMEM_SHARED((N_PAD, D), jnp.float32),
        pltpu.SemaphoreType.DMA,
    ],
)
def _deg_kernel(dst_hbm, ones_hbm, zeros_hbm, out_hbm, dst_v, ones_v, acc_sh,
                sem):
    cid = lax.axis_index("c")
    sid = lax.axis_index("s")
    wid = sid * NC + cid
    pltpu.sync_copy(zeros_hbm, acc_sh.at[pl.ds(sid * RPT, RPT)])
    pltpu.sync_copy(dst_hbm.at[wid], dst_v)
    pltpu.sync_copy(ones_hbm, ones_v)
    plsc.subcore_barrier()

    def body(j, carry):
        pltpu.async_copy(ones_v, acc_sh.at[dst_v.at[j]], sem, add=True)
        return carry

    lax.fori_loop(0, NCH, body, 0)

    def drain(j, carry):
        pltpu.make_async_copy(ones_v, acc_sh.at[dst_v.at[0]], sem).wait()
        return carry

    lax.fori_loop(0, NCH, drain, 0)
    plsc.subcore_barrier()
    pltpu.sync_copy(
        acc_sh.at[pl.ds(sid * RPT, RPT)],
        out_hbm.at[pl.ds(cid * N_PAD + sid * RPT, RPT)],
    )


@functools.partial(
    pl.kernel,
    mesh=_mesh,
    out_type=jax.ShapeDtypeStruct((NC * N_PAD, D), jnp.float32),
    scratch_types=[
        pltpu.VMEM((NCH, C), jnp.int32),
        pltpu.VMEM((8, C), jnp.int32),
        pltpu.VMEM((C, D), jnp.float32),
        pltpu.VMEM((C, D), jnp.float32),
        pltpu.VMEM_SHARED((N_PAD, D), jnp.float32),
        pltpu.SemaphoreType.DMA,
        pltpu.SemaphoreType.DMA,
        pltpu.SemaphoreType.DMA,
        pltpu.SemaphoreType.DMA,
        pltpu.SemaphoreType.DMA,
    ],
)
def _msg_kernel(u_hbm, src_hbm, dst_hbm, zeros_hbm, out_hbm,
                didx, sidx, b0, b1, acc_sh, isem, g0, g1, s0, s1):
    cid = lax.axis_index("c")
    sid = lax.axis_index("s")
    wid = sid * NC + cid
    bufs = (b0, b1)
    gsems = (g0, g1)
    ssems = (s0, s1)

    pltpu.sync_copy(zeros_hbm, acc_sh.at[pl.ds(sid * RPT, RPT)])
    pltpu.sync_copy(dst_hbm.at[wid], didx)

    def fetch_src(g):
        pltpu.async_copy(src_hbm.at[wid, g], sidx, isem)

    def wait_src():
        pltpu.make_async_copy(src_hbm.at[wid, 0], sidx, isem).wait()

    def sg(row, k):
        pltpu.async_copy(u_hbm.at[sidx.at[row]], bufs[k], gsems[k])

    def wg(k):
        pltpu.make_async_copy(u_hbm.at[sidx.at[0]], bufs[k], gsems[k]).wait()

    def ssc(j, k):
        pltpu.async_copy(bufs[k], acc_sh.at[didx.at[j]], ssems[k], add=True)

    def ws(k):
        pltpu.make_async_copy(bufs[k], acc_sh.at[didx.at[0]],
                              ssems[k]).wait()

    fetch_src(0)
    plsc.subcore_barrier()
    wait_src()
    sg(0, 0)
    sg(1, 1)

    def body(g, carry):
        base = g * 8
        for c in range(6):
            k = c % 2
            wg(k)
            ssc(base + c, k)
            ws(k)
            sg(c + 2, k)
        wg(0)
        ssc(base + 6, 0)
        wg(1)
        fetch_src(g + 1)
        ssc(base + 7, 1)
        ws(0)
        ws(1)
        wait_src()
        sg(0, 0)
        sg(1, 1)
        return carry

    lax.fori_loop(0, NG - 1, body, 0)
    base = (NG - 1) * 8
    for c in range(6):
        k = c % 2
        wg(k)
        ssc(base + c, k)
        ws(k)
        sg(c + 2, k)
    wg(0)
    ssc(base + 6, 0)
    wg(1)
    ssc(base + 7, 1)
    ws(0)
    ws(1)
    plsc.subcore_barrier()
    pltpu.sync_copy(
        acc_sh.at[pl.ds(sid * RPT, RPT)],
        out_hbm.at[pl.ds(cid * N_PAD + sid * RPT, RPT)],
    )


_BLK = 2000


def _mm_body(x_ref, w_ref, xw_ref):
    xw_ref[...] = jnp.dot(x_ref[...], w_ref[...],
                          preferred_element_type=jnp.float32)


def _scale_body(xw_ref, p0_ref, p1_ref, u_ref):
    deg = p0_ref[:, 0:1] + p1_ref[:, 0:1] + 1.0
    dinv = lax.rsqrt(deg)
    u_ref[...] = xw_ref[...] * dinv


def _final_body(x_ref, u_ref, s0_ref, s1_ref, p0_ref, p1_ref, b_ref, g_ref,
                bt_ref, h_ref):
    deg = p0_ref[:, 0:1] + p1_ref[:, 0:1] + 1.0
    dinv = lax.rsqrt(deg)
    z = dinv * (s0_ref[...] + s1_ref[...] + u_ref[...]) + b_ref[...]
    h = x_ref[...] + z
    mu = jnp.mean(h, axis=-1, keepdims=True)
    d = h - mu
    var = jnp.mean(d * d, axis=-1, keepdims=True)
    out = d * lax.rsqrt(var + 1e-5) * g_ref[...] + bt_ref[...]
    h_ref[...] = jnp.maximum(out, 0.0)


def kernel(x, edge_index, W, b, gamma, beta):
    pad = E_PAD - E
    r = jnp.arange(pad, dtype=edge_index.dtype)
    srcp = jnp.concatenate([edge_index[0], r % N])
    dstp = jnp.concatenate([edge_index[1], N + r % (N_PAD - N)])
    src4 = srcp.reshape(NW, NG, 8, C)
    dst3 = dstp.reshape(NW, NCH, C)

    onesD = jnp.ones((C, D), jnp.float32)
    zerosD = jnp.zeros((RPT, D), jnp.float32)

    deg_parts = _deg_kernel(dst3, onesD, zerosD)
    p0 = deg_parts[:N, :8]
    p1 = deg_parts[N_PAD:N_PAD + N, :8]

    nblk = N // _BLK
    row_spec = pl.BlockSpec((_BLK, D), lambda i: (i, 0))
    cnt_spec = pl.BlockSpec((_BLK, 8), lambda i: (i, 0))
    vec_spec = pl.BlockSpec((1, D), lambda i: (0, 0))

    xw = pl.pallas_call(
        _mm_body,
        grid=(nblk,),
        in_specs=[row_spec, pl.BlockSpec((D, D), lambda i: (0, 0))],
        out_specs=row_spec,
        out_shape=jax.ShapeDtypeStruct((N, D), jnp.float32),
    )(x, W)

    u = pl.pallas_call(
        _scale_body,
        grid=(nblk,),
        in_specs=[row_spec, cnt_spec, cnt_spec],
        out_specs=row_spec,
        out_shape=jax.ShapeDtypeStruct((N, D), jnp.float32),
    )(xw, p0, p1)

    s_parts = _msg_kernel(u, src4, dst3, zerosD)
    s0 = s_parts[:N]
    s1 = s_parts[N_PAD:N_PAD + N]

    h = pl.pallas_call(
        _final_body,
        grid=(nblk,),
        in_specs=[
            row_spec, row_spec, row_spec, row_spec, cnt_spec, cnt_spec,
            vec_spec, vec_spec, vec_spec,
        ],
        out_specs=row_spec,
        out_shape=jax.ShapeDtypeStruct((N, D), jnp.float32),
    )(x, u, s0, s1, p0, p1,
      b.reshape(1, D), gamma.reshape(1, D), beta.reshape(1, D))
    return h


# speedup vs baseline: 1.0039x; 1.0039x over previous
import functools

import jax
import jax.numpy as jnp
from jax import lax
from jax.experimental import pallas as pl
from jax.experimental.pallas import tpu as pltpu
from jax.experimental.pallas import tpu_sc as plsc

N = 10000
E = 320000
D = 128

NC = 2
NS = 16
NW = NC * NS
C = 128
NG = 10
NCH = 8 * NG
EPW = NCH * C
E_PAD = NW * EPW
N_PAD = 10240
RPT = N_PAD // NS

_mesh = plsc.VectorSubcoreMesh(core_axis_name="c", subcore_axis_name="s")


@functools.partial(
    pl.kernel,
    mesh=_mesh,
    out_type=jax.ShapeDtypeStruct((NC * N_PAD, D), jnp.float32),
    scratch_types=[
        pltpu.VMEM((NCH, C), jnp.int32),
        pltpu.VMEM((C, D), jnp.float32),
        pltpu.VMEM_SHARED((N_PAD, D), jnp.float32),
        pltpu.SemaphoreType.DMA,
    ],
)
def _deg_kernel(dst_hbm, ones_hbm, zeros_hbm, out_hbm, dst_v, ones_v, acc_sh,
                sem):
    cid = lax.axis_index("c")
    sid = lax.axis_index("s")
    wid = sid * NC + cid
    pltpu.sync_copy(zeros_hbm, acc_sh.at[pl.ds(sid * RPT, RPT)])
    pltpu.sync_copy(dst_hbm.at[wid], dst_v)
    pltpu.sync_copy(ones_hbm, ones_v)
    plsc.subcore_barrier()

    def body(j, carry):
        pltpu.async_copy(ones_v, acc_sh.at[dst_v.at[j]], sem, add=True)
        return carry

    lax.fori_loop(0, NCH, body, 0)

    def drain(j, carry):
        pltpu.make_async_copy(ones_v, acc_sh.at[dst_v.at[0]], sem).wait()
        return carry

    lax.fori_loop(0, NCH, drain, 0)
    plsc.subcore_barrier()
    pltpu.sync_copy(
        acc_sh.at[pl.ds(sid * RPT, RPT)],
        out_hbm.at[pl.ds(cid * N_PAD + sid * RPT, RPT)],
    )


@functools.partial(
    pl.kernel,
    mesh=_mesh,
    out_type=jax.ShapeDtypeStruct((NC * N_PAD, D), jnp.float32),
    scratch_types=[
        pltpu.VMEM((NCH, C), jnp.int32),
        pltpu.VMEM((8, C), jnp.int32),
        pltpu.VMEM((C, D), jnp.float32),
        pltpu.VMEM((C, D), jnp.float32),
        pltpu.VMEM_SHARED((N_PAD, D), jnp.float32),
        pltpu.SemaphoreType.DMA,
        pltpu.SemaphoreType.DMA,
        pltpu.SemaphoreType.DMA,
        pltpu.SemaphoreType.DMA,
        pltpu.SemaphoreType.DMA,
    ],
)
def _msg_kernel(u_hbm, src_hbm, dst_hbm, zeros_hbm, out_hbm,
                didx, sidx, b0, b1, acc_sh, isem, g0, g1, s0, s1):
    cid = lax.axis_index("c")
    sid = lax.axis_index("s")
    wid = sid * NC + cid
    bufs = (b0, b1)
    gsems = (g0, g1)
    ssems = (s0, s1)

    pltpu.sync_copy(zeros_hbm, acc_sh.at[pl.ds(sid * RPT, RPT)])
    pltpu.sync_copy(dst_hbm.at[wid], didx)

    def fetch_src(g):
        pltpu.async_copy(src_hbm.at[wid, g], sidx, isem)

    def wait_src():
        pltpu.make_async_copy(src_hbm.at[wid, 0], sidx, isem).wait()

    def sg(row, k):
        pltpu.async_copy(u_hbm.at[sidx.at[row]], bufs[k], gsems[k])

    def wg(k):
        pltpu.make_async_copy(u_hbm.at[sidx.at[0]], bufs[k], gsems[k]).wait()

    def ssc(j, k):
        pltpu.async_copy(bufs[k], acc_sh.at[didx.at[j]], ssems[k], add=True)

    def ws(k):
        pltpu.make_async_copy(bufs[k], acc_sh.at[didx.at[0]],
                              ssems[k]).wait()

    fetch_src(0)
    plsc.subcore_barrier()
    wait_src()
    sg(0, 0)
    sg(1, 1)

    def body(g, carry):
        base = g * 8
        for c in range(6):
            k = c % 2
            wg(k)
            ssc(base + c, k)
            ws(k)
            sg(c + 2, k)
        wg(0)
        ssc(base + 6, 0)
        wg(1)
        fetch_src(g + 1)
        ssc(base + 7, 1)
        ws(0)
        ws(1)
        wait_src()
        sg(0, 0)
        sg(1, 1)
        return carry

    lax.fori_loop(0, NG - 1, body, 0)
    base = (NG - 1) * 8
    for c in range(6):
        k = c % 2
        wg(k)
        ssc(base + c, k)
        ws(k)
        sg(c + 2, k)
    wg(0)
    ssc(base + 6, 0)
    wg(1)
    ssc(base + 7, 1)
    ws(0)
    ws(1)
    plsc.subcore_barrier()
    pltpu.sync_copy(
        acc_sh.at[pl.ds(sid * RPT, RPT)],
        out_hbm.at[pl.ds(cid * N_PAD + sid * RPT, RPT)],
    )


_BLK = 2000


def _scale_body(x_ref, w_ref, p0_ref, p1_ref, u_ref):
    deg = p0_ref[:, 0:1] + p1_ref[:, 0:1] + 1.0
    dinv = lax.rsqrt(deg)
    xw = jnp.dot(x_ref[...], w_ref[...], preferred_element_type=jnp.float32)
    u_ref[...] = xw * dinv


def _final_body(x_ref, u_ref, s0_ref, s1_ref, p0_ref, p1_ref, b_ref, g_ref,
                bt_ref, h_ref):
    deg = p0_ref[:, 0:1] + p1_ref[:, 0:1] + 1.0
    dinv = lax.rsqrt(deg)
    z = dinv * (s0_ref[...] + s1_ref[...] + u_ref[...]) + b_ref[...]
    h = x_ref[...] + z
    mu = jnp.mean(h, axis=-1, keepdims=True)
    d = h - mu
    var = jnp.mean(d * d, axis=-1, keepdims=True)
    out = d * lax.rsqrt(var + 1e-5) * g_ref[...] + bt_ref[...]
    h_ref[...] = jnp.maximum(out, 0.0)


def kernel(x, edge_index, W, b, gamma, beta):
    pad = E_PAD - E
    r = jnp.arange(pad, dtype=edge_index.dtype)
    srcp = jnp.concatenate([edge_index[0], r % N])
    dstp = jnp.concatenate([edge_index[1], N + r % (N_PAD - N)])
    src4 = srcp.reshape(NW, NG, 8, C)
    dst3 = dstp.reshape(NW, NCH, C)

    onesD = jnp.ones((C, D), jnp.float32)
    zerosD = jnp.zeros((RPT, D), jnp.float32)

    deg_parts = _deg_kernel(dst3, onesD, zerosD)
    p0 = deg_parts[:N, :8]
    p1 = deg_parts[N_PAD:N_PAD + N, :8]

    nblk = N // _BLK
    row_spec = pl.BlockSpec((_BLK, D), lambda i: (i, 0))
    cnt_spec = pl.BlockSpec((_BLK, 8), lambda i: (i, 0))
    vec_spec = pl.BlockSpec((1, D), lambda i: (0, 0))

    u = pl.pallas_call(
        _scale_body,
        grid=(nblk,),
        in_specs=[row_spec, pl.BlockSpec((D, D), lambda i: (0, 0)),
                  cnt_spec, cnt_spec],
        out_specs=row_spec,
        out_shape=jax.ShapeDtypeStruct((N, D), jnp.float32),
    )(x, W, p0, p1)

    s_parts = _msg_kernel(u, src4, dst3, zerosD)
    s0 = s_parts[:N]
    s1 = s_parts[N_PAD:N_PAD + N]

    h = pl.pallas_call(
        _final_body,
        grid=(nblk,),
        in_specs=[
            row_spec, row_spec, row_spec, row_spec, cnt_spec, cnt_spec,
            vec_spec, vec_spec, vec_spec,
        ],
        out_specs=row_spec,
        out_shape=jax.ShapeDtypeStruct((N, D), jnp.float32),
    )(x, u, s0, s1, p0, p1,
      b.reshape(1, D), gamma.reshape(1, D), beta.reshape(1, D))
    return h
